# speedup vs baseline: 4.0976x; 4.0976x over previous
import functools

import jax
import jax.numpy as jnp
from jax import lax
from jax.experimental import pallas as pl
from jax.experimental.pallas import tpu as pltpu
from jax.experimental.pallas import tpu_sc as plsc

N = 10000
E = 320000
D = 128
DE = 16

NC = 2
NS = 16
NW = NC * NS
C = 128

N_PAD = 10240
ROWS_PER_TILE = N_PAD // NS

BE = 2560


def _xw_body(x_ref, w_ref, b_ref, o_ref):
    o_ref[...] = (
        jnp.dot(x_ref[...], w_ref[...], preferred_element_type=jnp.float32)
        + b_ref[...]
    )


def _edge_body(g_ref, ea_ref, w_ref, g1_ref, bt1_ref, o_ref):
    f = g_ref[...] + jnp.dot(
        ea_ref[...], w_ref[...], preferred_element_type=jnp.float32
    )
    m = jnp.mean(f, axis=-1, keepdims=True)
    cgap = f - m
    v = jnp.mean(cgap * cgap, axis=-1, keepdims=True)
    h = cgap * lax.rsqrt(v + 1e-5) * g1_ref[...] + bt1_ref[...]
    o_ref[...] = jnp.maximum(h, 0.0)


def _node_body(p_ref, wn_ref, bn_ref, gn_ref, btn_ref, o_ref):
    ft = p_ref[0, :N, :] + p_ref[1, :N, :]
    f = jnp.dot(ft, wn_ref[...], preferred_element_type=jnp.float32) + bn_ref[...]
    m = jnp.mean(f, axis=-1, keepdims=True)
    cgap = f - m
    v = jnp.mean(cgap * cgap, axis=-1, keepdims=True)
    h = cgap * lax.rsqrt(v + 1e-5) * gn_ref[...] + btn_ref[...]
    o_ref[...] = jnp.maximum(h, 0.0)


_sc_mesh = plsc.VectorSubcoreMesh(core_axis_name="c", subcore_axis_name="s")


@functools.partial(
    pl.kernel,
    out_type=jax.ShapeDtypeStruct((E, D), jnp.float32),
    mesh=_sc_mesh,
    scratch_types=[
        pltpu.VMEM((C,), jnp.int32),
        pltpu.VMEM((C, D), jnp.float32),
        pltpu.SemaphoreType.DMA,
    ],
)
def _gather_sc(xw_hbm, col_hbm, g_hbm, idx_v, rows_v, sem):
    wid = lax.axis_index("s") * NC + lax.axis_index("c")
    nchunks = E // C

    def body(j, carry):
        chunk = wid + j * NW

        @pl.when(chunk < nchunks)
        def _():
            base = chunk * C
            pltpu.sync_copy(col_hbm.at[pl.ds(base, C)], idx_v)
            pltpu.async_copy(xw_hbm.at[idx_v], rows_v, sem).wait()
            pltpu.sync_copy(rows_v, g_hbm.at[pl.ds(base, C)])

        return carry

    lax.fori_loop(0, (nchunks + NW - 1) // NW, body, 0)


@functools.partial(
    pl.kernel,
    out_type=jax.ShapeDtypeStruct((NC, N_PAD, D), jnp.float32),
    mesh=_sc_mesh,
    scratch_types=[
        pltpu.VMEM((2, C), jnp.int32),
        pltpu.VMEM((C, D), jnp.float32),
        pltpu.VMEM_SHARED((N_PAD, D), jnp.float32),
        pltpu.SemaphoreType.DMA,
    ],
)
def _scatter_sc(f_hbm, row_hbm, col_hbm, zeros_hbm, out_hbm, idx_v, fbuf, flow_sh, sem):
    cid = lax.axis_index("c")
    sid = lax.axis_index("s")

    pltpu.sync_copy(zeros_hbm, flow_sh.at[pl.ds(sid * ROWS_PER_TILE, ROWS_PER_TILE)])
    plsc.subcore_barrier()

    nchunks_half = (E // C) // NC

    def body(j, carry):
        chunk_local = sid + j * NS

        @pl.when(chunk_local < nchunks_half)
        def _():
            base = (cid * nchunks_half + chunk_local) * C
            pltpu.sync_copy(row_hbm.at[pl.ds(base, C)], idx_v.at[0])
            pltpu.sync_copy(col_hbm.at[pl.ds(base, C)], idx_v.at[1])
            for i in range(C // 16):
                r = idx_v[0, pl.ds(i * 16, 16)]
                cc = idx_v[1, pl.ds(i * 16, 16)]
                trash = jnp.full((16,), N, jnp.int32)
                idx_v[0, pl.ds(i * 16, 16)] = jnp.where(r == cc, trash, r)
            pltpu.sync_copy(f_hbm.at[pl.ds(base, C)], fbuf)
            pltpu.sync_copy(fbuf, flow_sh.at[idx_v.at[0]], add=True)

        return carry

    lax.fori_loop(0, (nchunks_half + NS - 1) // NS, body, 0)
    plsc.subcore_barrier()
    pltpu.sync_copy(
        flow_sh.at[pl.ds(sid * ROWS_PER_TILE, ROWS_PER_TILE)],
        out_hbm.at[cid, pl.ds(sid * ROWS_PER_TILE, ROWS_PER_TILE)],
    )


def kernel(x, edge_index, edge_attr, W1, b1, g1, bt1, Wn, bn, gn, btn):
    row = edge_index[0]
    col = edge_index[1]
    W1a = W1[:D]
    W1b = W1[D:]

    xw = pl.pallas_call(
        _xw_body,
        out_shape=jax.ShapeDtypeStruct((N, D), jnp.float32),
    )(x, W1a, b1.reshape(1, D))

    g = _gather_sc(xw, col)

    nblk = E // BE
    f = pl.pallas_call(
        _edge_body,
        grid=(nblk,),
        in_specs=[
            pl.BlockSpec((BE, D), lambda i: (i, 0)),
            pl.BlockSpec((BE, DE), lambda i: (i, 0)),
            pl.BlockSpec((DE, D), lambda i: (0, 0)),
            pl.BlockSpec((1, D), lambda i: (0, 0)),
            pl.BlockSpec((1, D), lambda i: (0, 0)),
        ],
        out_specs=pl.BlockSpec((BE, D), lambda i: (i, 0)),
        out_shape=jax.ShapeDtypeStruct((E, D), jnp.float32),
    )(g, edge_attr, W1b, g1.reshape(1, D), bt1.reshape(1, D))

    zeros = jnp.zeros((ROWS_PER_TILE, D), jnp.float32)
    partials = _scatter_sc(f, row, col, zeros)

    out = pl.pallas_call(
        _node_body,
        out_shape=jax.ShapeDtypeStruct((N, D), jnp.float32),
    )(partials, Wn, bn.reshape(1, D), gn.reshape(1, D), btn.reshape(1, D))
    return out


# speedup vs baseline: 5.1865x; 1.2657x over previous
import functools

import jax
import jax.numpy as jnp
from jax import lax
from jax.experimental import pallas as pl
from jax.experimental.pallas import tpu as pltpu
from jax.experimental.pallas import tpu_sc as plsc

N = 10000
E = 320000
D = 128
DE = 16

NC = 2
NS = 16
NW = NC * NS
C = 128

N_PAD = 10240
ROWS_PER_TILE = N_PAD // NS

BE = 2560


def _xw_body(x_ref, w_ref, b_ref, o_ref):
    o_ref[...] = (
        jnp.dot(x_ref[...], w_ref[...], preferred_element_type=jnp.float32)
        + b_ref[...]
    )


def _edge_body(g_ref, ea_ref, w_ref, g1_ref, bt1_ref, o_ref):
    f = g_ref[...] + jnp.dot(
        ea_ref[...], w_ref[...], preferred_element_type=jnp.float32
    )
    m = jnp.mean(f, axis=-1, keepdims=True)
    cgap = f - m
    v = jnp.mean(cgap * cgap, axis=-1, keepdims=True)
    h = cgap * lax.rsqrt(v + 1e-5) * g1_ref[...] + bt1_ref[...]
    o_ref[...] = jnp.maximum(h, 0.0)


def _node_body(p_ref, wn_ref, bn_ref, gn_ref, btn_ref, o_ref):
    ft = p_ref[0, :N, :] + p_ref[1, :N, :]
    f = jnp.dot(ft, wn_ref[...], preferred_element_type=jnp.float32) + bn_ref[...]
    m = jnp.mean(f, axis=-1, keepdims=True)
    cgap = f - m
    v = jnp.mean(cgap * cgap, axis=-1, keepdims=True)
    h = cgap * lax.rsqrt(v + 1e-5) * gn_ref[...] + btn_ref[...]
    o_ref[...] = jnp.maximum(h, 0.0)


_sc_mesh = plsc.VectorSubcoreMesh(core_axis_name="c", subcore_axis_name="s")

K = 6
NGRP = 13
KS = 2
NGRPS = 39


@functools.partial(
    pl.kernel,
    out_type=jax.ShapeDtypeStruct((E, D), jnp.float32),
    mesh=_sc_mesh,
    scratch_types=[
        pltpu.VMEM((K, C), jnp.int32),
        pltpu.VMEM((K, C, D), jnp.float32),
        pltpu.SemaphoreType.DMA((K,)),
        pltpu.SemaphoreType.DMA((K,)),
        pltpu.SemaphoreType.DMA((K,)),
    ],
)
def _gather_sc(xw_hbm, col_hbm, g_hbm, idx_v, rows_v, sem_i, sem_g, sem_s):
    wid = lax.axis_index("s") * NC + lax.axis_index("c")
    nchunks = E // C

    def body(grp, carry):
        for i in range(K):
            chunk = wid + (grp * K + i) * NW
            pltpu.async_copy(
                col_hbm.at[pl.ds(chunk * C, C)], idx_v.at[i], sem_i.at[i]
            )
        for i in range(K):
            @pl.when(grp > 0)
            def _():
                pltpu.make_async_copy(
                    rows_v.at[i], g_hbm.at[pl.ds(0, C)], sem_s.at[i]
                ).wait()
        for i in range(K):
            pltpu.make_async_copy(
                col_hbm.at[pl.ds(0, C)], idx_v.at[i], sem_i.at[i]
            ).wait()
            pltpu.async_copy(xw_hbm.at[idx_v.at[i]], rows_v.at[i], sem_g.at[i])
        for i in range(K):
            chunk = wid + (grp * K + i) * NW
            pltpu.make_async_copy(
                xw_hbm.at[idx_v.at[i]], rows_v.at[i], sem_g.at[i]
            ).wait()
            pltpu.async_copy(
                rows_v.at[i], g_hbm.at[pl.ds(chunk * C, C)], sem_s.at[i]
            )
        return carry

    lax.fori_loop(0, NGRP, body, 0)
    for i in range(K):
        pltpu.make_async_copy(
            rows_v.at[i], g_hbm.at[pl.ds(0, C)], sem_s.at[i]
        ).wait()

    @pl.when(wid < nchunks - NGRP * K * NW)
    def _():
        base = (NGRP * K * NW + wid) * C
        pltpu.sync_copy(col_hbm.at[pl.ds(base, C)], idx_v.at[0])
        pltpu.async_copy(xw_hbm.at[idx_v.at[0]], rows_v.at[0], sem_g.at[0]).wait()
        pltpu.sync_copy(rows_v.at[0], g_hbm.at[pl.ds(base, C)])


@functools.partial(
    pl.kernel,
    out_type=jax.ShapeDtypeStruct((NC, N_PAD, D), jnp.float32),
    mesh=_sc_mesh,
    scratch_types=[
        pltpu.VMEM((2 * KS, C), jnp.int32),
        pltpu.VMEM((KS, C, D), jnp.float32),
        pltpu.VMEM_SHARED((N_PAD, D), jnp.float32),
        pltpu.SemaphoreType.DMA((KS,)),
        pltpu.SemaphoreType.DMA((KS,)),
        pltpu.SemaphoreType.DMA((KS,)),
    ],
)
def _scatter_sc(
    f_hbm, row_hbm, col_hbm, zeros_hbm, out_hbm,
    idx_v, fbuf, flow_sh, sem_i, sem_f, sem_sc,
):
    cid = lax.axis_index("c")
    sid = lax.axis_index("s")

    pltpu.sync_copy(zeros_hbm, flow_sh.at[pl.ds(sid * ROWS_PER_TILE, ROWS_PER_TILE)])
    plsc.subcore_barrier()

    nchunks_half = (E // C) // NC

    def _select_trash(i):
        for ii in range(C // 16):
            r = idx_v[2 * i, pl.ds(ii * 16, 16)]
            cc = idx_v[2 * i + 1, pl.ds(ii * 16, 16)]
            trash = jnp.full((16,), N, jnp.int32)
            idx_v[2 * i, pl.ds(ii * 16, 16)] = jnp.where(r == cc, trash, r)

    def body(grp, carry):
        for i in range(KS):
            @pl.when(grp > 0)
            def _():
                pltpu.make_async_copy(
                    fbuf.at[i], flow_sh.at[idx_v.at[2 * i]], sem_sc.at[i]
                ).wait()
            chunk_local = sid + (grp * KS + i) * NS
            base = (cid * nchunks_half + chunk_local) * C
            pltpu.async_copy(row_hbm.at[pl.ds(base, C)], idx_v.at[2 * i], sem_i.at[i])
            pltpu.async_copy(col_hbm.at[pl.ds(base, C)], idx_v.at[2 * i + 1], sem_i.at[i])
            pltpu.async_copy(f_hbm.at[pl.ds(base, C)], fbuf.at[i], sem_f.at[i])
        for i in range(KS):
            pltpu.make_async_copy(
                row_hbm.at[pl.ds(0, C)], idx_v.at[2 * i], sem_i.at[i]
            ).wait()
            pltpu.make_async_copy(
                col_hbm.at[pl.ds(0, C)], idx_v.at[2 * i + 1], sem_i.at[i]
            ).wait()
            _select_trash(i)
            pltpu.make_async_copy(
                f_hbm.at[pl.ds(0, C)], fbuf.at[i], sem_f.at[i]
            ).wait()
            pltpu.async_copy(
                fbuf.at[i], flow_sh.at[idx_v.at[2 * i]], sem_sc.at[i], add=True
            )
        return carry

    lax.fori_loop(0, NGRPS, body, 0)
    for i in range(KS):
        pltpu.make_async_copy(
            fbuf.at[i], flow_sh.at[idx_v.at[2 * i]], sem_sc.at[i]
        ).wait()

    @pl.when(sid < nchunks_half - NGRPS * KS * NS)
    def _():
        base = (cid * nchunks_half + NGRPS * KS * NS + sid) * C
        pltpu.sync_copy(row_hbm.at[pl.ds(base, C)], idx_v.at[0])
        pltpu.sync_copy(col_hbm.at[pl.ds(base, C)], idx_v.at[1])
        _select_trash(0)
        pltpu.sync_copy(f_hbm.at[pl.ds(base, C)], fbuf.at[0])
        pltpu.sync_copy(fbuf.at[0], flow_sh.at[idx_v.at[0]], add=True)

    plsc.subcore_barrier()
    pltpu.sync_copy(
        flow_sh.at[pl.ds(sid * ROWS_PER_TILE, ROWS_PER_TILE)],
        out_hbm.at[cid, pl.ds(sid * ROWS_PER_TILE, ROWS_PER_TILE)],
    )


def kernel(x, edge_index, edge_attr, W1, b1, g1, bt1, Wn, bn, gn, btn):
    row = edge_index[0]
    col = edge_index[1]
    W1a = W1[:D]
    W1b = W1[D:]

    xw = pl.pallas_call(
        _xw_body,
        out_shape=jax.ShapeDtypeStruct((N, D), jnp.float32),
    )(x, W1a, b1.reshape(1, D))

    g = _gather_sc(xw, col)

    nblk = E // BE
    f = pl.pallas_call(
        _edge_body,
        grid=(nblk,),
        in_specs=[
            pl.BlockSpec((BE, D), lambda i: (i, 0)),
            pl.BlockSpec((BE, DE), lambda i: (i, 0)),
            pl.BlockSpec((DE, D), lambda i: (0, 0)),
            pl.BlockSpec((1, D), lambda i: (0, 0)),
            pl.BlockSpec((1, D), lambda i: (0, 0)),
        ],
        out_specs=pl.BlockSpec((BE, D), lambda i: (i, 0)),
        out_shape=jax.ShapeDtypeStruct((E, D), jnp.float32),
    )(g, edge_attr, W1b, g1.reshape(1, D), bt1.reshape(1, D))

    zeros = jnp.zeros((ROWS_PER_TILE, D), jnp.float32)
    partials = _scatter_sc(f, row, col, zeros)

    out = pl.pallas_call(
        _node_body,
        out_shape=jax.ShapeDtypeStruct((N, D), jnp.float32),
    )(partials, Wn, bn.reshape(1, D), gn.reshape(1, D), btn.reshape(1, D))
    return out
